# baseline (speedup 1.0000x reference)
import jax
import jax.numpy as jnp
from jax import lax
from jax.experimental import pallas as pl
from jax.experimental.pallas import tpu as pltpu
from jax.experimental.pallas import tpu_sc as plsc

NC = 2
NS = 16
NW = NC * NS

B = 16384
S = 50
V = 1_000_000
D = 32

TC = V // 128
TC_MAIN = 7808
TAIL_C0 = 999936
NIT = (S + 7) // 8 * (B // 128)
NIT_PW = NIT // NW

BPW = B // NW
NSPAIR = S // 2


def _iota16():
    return lax.iota(jnp.int32, 16)


def _p1_body(wt, idxT, tailp, wf, idxf, ibuf, obuf, itile, gi0, gi1, go0, go1, si, so):
    w = lax.axis_index("s") * NC + lax.axis_index("c")
    iota = _iota16()

    def fire_in(c, p):
        pltpu.async_copy(wt.at[:, pl.ds(c * 128, 128)], ibuf.at[p], gi0 if p == 0 else gi1)

    def wait_in(p):
        pltpu.make_async_copy(wt.at[:, pl.ds(0, 128)], ibuf.at[p], gi0 if p == 0 else gi1).wait()

    def fire_out(c, p):
        pltpu.async_copy(obuf.at[p], wf.at[pl.ds(c * 32, 32), :], go0 if p == 0 else go1)

    def wait_out(p):
        pltpu.make_async_copy(obuf.at[p], wf.at[pl.ds(0, 32), :], go0 if p == 0 else go1).wait()

    def permute(p, ncols):
        pv = jnp.full((16,), p, jnp.int32)
        for cin in range(ncols):
            cv = jnp.full((16,), cin, jnp.int32)
            x0 = plsc.load_gather(ibuf, [pv, iota, cv])
            x1 = plsc.load_gather(ibuf, [pv, iota + 16, cv])
            r, off = cin // 4, (cin % 4) * 32
            obuf[p, r, pl.ds(off, 16)] = x0
            obuf[p, r, pl.ds(off + 16, 16)] = x1

    c_of = lambda k: w + NW * k
    fire_in(c_of(0), 0)

    def pair(k2, carry):
        k0 = 2 * k2
        wait_in(0)
        fire_in(c_of(k0 + 1), 1)

        @pl.when(k2 > 0)
        def _():
            wait_out(0)

        permute(0, 128)
        fire_out(c_of(k0), 0)
        wait_in(1)

        @pl.when(k2 < NIT_PAIRS - 1)
        def _():
            fire_in(c_of(k0 + 2), 0)

        @pl.when(k2 > 0)
        def _():
            wait_out(1)

        permute(1, 128)
        fire_out(c_of(k0 + 1), 1)
        return carry

    NIT_PAIRS = 122
    lax.fori_loop(0, NIT_PAIRS, pair, 0)
    wait_out(0)
    wait_out(1)

    @pl.when(w < 4)
    def _():
        c = TC_MAIN + w
        pltpu.sync_copy(wt.at[:, pl.ds(c * 128, 128)], ibuf.at[0])
        permute(0, 128)
        pltpu.sync_copy(obuf.at[0], wf.at[pl.ds(c * 32, 32), :])

    @pl.when(w == NW - 1)
    def _():
        pltpu.sync_copy(tailp, ibuf.at[0])
        permute(0, 64)
        pltpu.sync_copy(obuf.at[0, pl.ds(0, 16), :], wf.at[pl.ds(TAIL_C0 // 4, 16), :])

    def it_fire_in(k, carry):
        t = w + NW * k
        tr = t // 128
        tc = t - tr * 128
        pltpu.async_copy(idxT.at[pl.ds(8 * tr, 8), pl.ds(128 * tc, 128)], itile.at[k], si)
        return carry

    lax.fori_loop(0, 24, it_fire_in, 0)
    for k in range(24, NIT_PW):
        pltpu.async_copy(idxT.at[pl.ds(48, 2), pl.ds(128 * (w + NW * k - 768), 128)],
                         itile.at[k, pl.ds(0, 2)], si)

    def it_drain_in(k, carry):
        pltpu.make_async_copy(idxT.at[pl.ds(0, 8), pl.ds(0, 128)], itile.at[k], si).wait()
        return carry

    lax.fori_loop(0, 24, it_drain_in, 0)
    for k in range(24, NIT_PW):
        pltpu.make_async_copy(idxT.at[pl.ds(0, 2), pl.ds(0, 128)],
                              itile.at[k, pl.ds(0, 2)], si).wait()

    def it_fire_out(k, carry):
        t = w + NW * k
        tr = t // 128
        tc = t - tr * 128
        for e in range(8):
            pltpu.async_copy(itile.at[k, e], idxf.at[(8 * tr + e) * 128 + tc, :], so)
        return carry

    lax.fori_loop(0, 24, it_fire_out, 0)
    for k in range(24, NIT_PW):
        tc = w + NW * k - 768
        for e in range(2):
            pltpu.async_copy(itile.at[k, e], idxf.at[(48 + e) * 128 + tc, :], so)

    def it_drain_out(k, carry):
        pltpu.make_async_copy(itile.at[0], idxf.at[pl.ds(0, 8), :], so).wait()
        return carry

    lax.fori_loop(0, 24, it_drain_out, 0)
    for k in range(24, NIT_PW):
        pltpu.make_async_copy(itile.at[0, pl.ds(0, 2)], idxf.at[pl.ds(0, 2), :], so).wait()


def _p2_body(idx3, table, out3, idx_v, rows0, rows1, tbuf0, tbuf1, g0, g1, o0, o1):
    w = lax.axis_index("s") * NC + lax.axis_index("c")
    iota = _iota16()
    b0 = w * BPW

    pltpu.sync_copy(idx3.at[:, pl.ds(4 * w, 4), :], idx_v)

    def fire_g(s, rows, sem):
        for j in range(4):
            pltpu.async_copy(table.at[idx_v.at[s, j]], rows.at[pl.ds(128 * j, 128)], sem)

    def wait_g(rows, sem):
        pltpu.make_async_copy(table.at[pl.ds(0, BPW)], rows, sem).wait()

    def fire_o(s, tbuf, sem):
        pltpu.async_copy(tbuf, out3.at[s, :, pl.ds(b0, BPW)], sem)

    def wait_o(tbuf, sem):
        pltpu.make_async_copy(tbuf, out3.at[0, :, pl.ds(0, BPW)], sem).wait()

    def transpose(rows, tbuf):
        dcols = [jnp.full((16,), d, jnp.int32) for d in range(D)]

        def vbody(v, carry):
            rv = 16 * v + iota
            for d in range(D):
                x = plsc.load_gather(rows, [rv, dcols[d]])
                tbuf[d, pl.ds(16 * v, 16)] = x
            return carry

        lax.fori_loop(0, BPW // 16, vbody, 0)

    fire_g(0, rows0, g0)

    def pair(s2, carry):
        s0 = 2 * s2
        s1 = s0 + 1
        wait_g(rows0, g0)
        fire_g(s1, rows1, g1)

        @pl.when(s2 > 0)
        def _():
            wait_o(tbuf0, o0)

        transpose(rows0, tbuf0)
        fire_o(s0, tbuf0, o0)
        wait_g(rows1, g1)

        @pl.when(s2 < NSPAIR - 1)
        def _():
            fire_g(s0 + 2, rows0, g0)

        @pl.when(s2 > 0)
        def _():
            wait_o(tbuf1, o1)

        transpose(rows1, tbuf1)
        fire_o(s1, tbuf1, o1)
        return carry

    lax.fori_loop(0, NSPAIR, pair, 0)
    wait_o(tbuf0, o0)
    wait_o(tbuf1, o1)


def kernel(inputs, index, weights):
    wt = weights.T
    idxT = inputs.T
    tailp = jnp.pad(weights[V - 64:].T, ((0, 0), (0, 64)))

    mesh = plsc.VectorSubcoreMesh(core_axis_name="c", subcore_axis_name="s")

    p1 = pl.kernel(
        _p1_body,
        out_type=[
            jax.ShapeDtypeStruct((V // 4, 128), jnp.float32),
            jax.ShapeDtypeStruct((B * S // 128, 128), jnp.int32),
        ],
        mesh=mesh,
        compiler_params=pltpu.CompilerParams(use_tc_tiling_on_sc=True,
                                             needs_layout_passes=False),
        scratch_types=[
            pltpu.VMEM((2, D, 128), jnp.float32),
            pltpu.VMEM((2, D, 128), jnp.float32),
            pltpu.VMEM((NIT_PW, 8, 128), jnp.int32),
            pltpu.SemaphoreType.DMA,
            pltpu.SemaphoreType.DMA,
            pltpu.SemaphoreType.DMA,
            pltpu.SemaphoreType.DMA,
            pltpu.SemaphoreType.DMA,
            pltpu.SemaphoreType.DMA,
        ],
    )
    wflat, idxf = p1(wt, idxT, tailp)

    table = wflat.reshape(V, D)
    idx3 = idxf.reshape(S, B // 128, 128)

    p2 = pl.kernel(
        _p2_body,
        out_type=jax.ShapeDtypeStruct((S, D, B), jnp.float32),
        mesh=mesh,
        compiler_params=pltpu.CompilerParams(use_tc_tiling_on_sc=False,
                                             needs_layout_passes=False),
        scratch_types=[
            pltpu.VMEM((S, 4, 128), jnp.int32),
            pltpu.VMEM((BPW, D), jnp.float32),
            pltpu.VMEM((BPW, D), jnp.float32),
            pltpu.VMEM((D, BPW), jnp.float32),
            pltpu.VMEM((D, BPW), jnp.float32),
            pltpu.SemaphoreType.DMA,
            pltpu.SemaphoreType.DMA,
            pltpu.SemaphoreType.DMA,
            pltpu.SemaphoreType.DMA,
        ],
    )
    out3 = p2(idx3, table)

    return out3.transpose(2, 0, 1)


# speedup vs baseline: 1.4630x; 1.4630x over previous
import jax
import jax.numpy as jnp
from jax import lax
from jax.experimental import pallas as pl
from jax.experimental.pallas import tpu as pltpu
from jax.experimental.pallas import tpu_sc as plsc

NC = 2
NS = 16
NW = NC * NS

B = 16384
S = 50
V = 1_000_000
D = 32

BPW = B // NW
NSPAIR = S // 2


def _iota16():
    return lax.iota(jnp.int32, 16)


def _p2_body(idx3, table, out_sb, idx_v, rows0, rows1, g0, g1, o0, o1):
    w = lax.axis_index("s") * NC + lax.axis_index("c")
    b0 = w * BPW

    pltpu.sync_copy(idx3.at[:, pl.ds(4 * w, 4), :], idx_v)

    def fire_g(s, rows, sem):
        for j in range(4):
            pltpu.async_copy(table.at[idx_v.at[s, j]],
                             rows.at[pl.ds(128 * j, 128)], sem)

    def wait_g(rows, sem):
        pltpu.make_async_copy(table.at[pl.ds(0, BPW)], rows, sem).wait()

    def fire_o(s, rows, sem):
        pltpu.async_copy(rows, out_sb.at[s, pl.ds(b0, BPW), :], sem)

    def wait_o(rows, sem):
        pltpu.make_async_copy(rows, out_sb.at[0, pl.ds(0, BPW), :], sem).wait()

    fire_g(0, rows0, g0)

    def pair(s2, carry):
        s0 = 2 * s2
        s1 = s0 + 1
        wait_g(rows0, g0)

        @pl.when(s2 > 0)
        def _():
            wait_o(rows1, o1)

        fire_g(s1, rows1, g1)
        fire_o(s0, rows0, o0)
        wait_g(rows1, g1)
        wait_o(rows0, o0)

        @pl.when(s2 < NSPAIR - 1)
        def _():
            fire_g(s0 + 2, rows0, g0)

        fire_o(s1, rows1, o1)
        return carry

    lax.fori_loop(0, NSPAIR, pair, 0)
    wait_o(rows1, o1)


def kernel(inputs, index, weights):
    wflat = lax.optimization_barrier(weights.reshape(V // 4, 128))
    table = wflat.reshape(V, D)
    idxf = lax.optimization_barrier(inputs.T.reshape(B * S // 128, 128))
    idx3 = idxf.reshape(S, B // 128, 128)

    mesh = plsc.VectorSubcoreMesh(core_axis_name="c", subcore_axis_name="s")

    p2 = pl.kernel(
        _p2_body,
        out_type=jax.ShapeDtypeStruct((S, B, D), jnp.float32),
        mesh=mesh,
        compiler_params=pltpu.CompilerParams(use_tc_tiling_on_sc=False,
                                             needs_layout_passes=False),
        scratch_types=[
            pltpu.VMEM((S, 4, 128), jnp.int32),
            pltpu.VMEM((BPW, D), jnp.float32),
            pltpu.VMEM((BPW, D), jnp.float32),
            pltpu.SemaphoreType.DMA,
            pltpu.SemaphoreType.DMA,
            pltpu.SemaphoreType.DMA,
            pltpu.SemaphoreType.DMA,
        ],
    )
    out_sb = p2(idx3, table)

    return out_sb.transpose(1, 0, 2)


# speedup vs baseline: 1.8967x; 1.2965x over previous
import jax
import jax.numpy as jnp
from jax import lax
from jax.experimental import pallas as pl
from jax.experimental.pallas import tpu as pltpu
from jax.experimental.pallas import tpu_sc as plsc

NC = 2
NS = 16
NW = NC * NS

B = 16384
S = 50
V = 1_000_000
D = 32

BPW = B // NW
NSPAIR = S // 2


def _iota16():
    return lax.iota(jnp.int32, 16)


def _p2_body(idx3, table, out3, idx_v, rows0, rows1, tbuf, obuf0, obuf1,
             g0, g1, o0, o1):
    w = lax.axis_index("s") * NC + lax.axis_index("c")
    iota = _iota16()
    b0 = w * BPW

    pltpu.sync_copy(idx3.at[:, pl.ds(4 * w, 4), :], idx_v)

    def fire_g(s, rows, sem):
        for j in range(4):
            pltpu.async_copy(table.at[idx_v.at[s, j]],
                             rows.at[pl.ds(128 * j, 128)], sem)

    def wait_g(rows, sem):
        pltpu.make_async_copy(table.at[pl.ds(0, BPW)], rows, sem).wait()

    def fire_o(s, obuf, sem):
        pltpu.async_copy(obuf, out3.at[s, :, pl.ds(b0, BPW)], sem)

    def wait_o(obuf, sem):
        pltpu.make_async_copy(obuf, out3.at[0, :, pl.ds(0, BPW)], sem).wait()

    def transpose(rows, obuf):
        def p1body(b2, carry):
            for u in range(8):
                bq = 8 * b2 + u
                bqv = jnp.full((16,), bq, jnp.int32)
                x0 = rows[bq, pl.ds(0, 16)]
                x1 = rows[bq, pl.ds(16, 16)]
                plsc.store_scatter(tbuf, [iota, bqv], x0)
                plsc.store_scatter(tbuf, [iota + 16, bqv], x1)
            return carry

        lax.fori_loop(0, BPW // 8, p1body, 0)

        def p2body(v, carry):
            for d in range(D):
                obuf[d, pl.ds(16 * v, 16)] = tbuf[d, pl.ds(16 * v, 16)]
            return carry

        lax.fori_loop(0, BPW // 16, p2body, 0)

    fire_g(0, rows0, g0)

    def pair(s2, carry):
        s0 = 2 * s2
        s1 = s0 + 1
        wait_g(rows0, g0)
        fire_g(s1, rows1, g1)

        @pl.when(s2 > 0)
        def _():
            wait_o(obuf0, o0)

        transpose(rows0, obuf0)
        fire_o(s0, obuf0, o0)
        wait_g(rows1, g1)

        @pl.when(s2 < NSPAIR - 1)
        def _():
            fire_g(s0 + 2, rows0, g0)

        @pl.when(s2 > 0)
        def _():
            wait_o(obuf1, o1)

        transpose(rows1, obuf1)
        fire_o(s1, obuf1, o1)
        return carry

    lax.fori_loop(0, NSPAIR, pair, 0)
    wait_o(obuf0, o0)
    wait_o(obuf1, o1)


def kernel(inputs, index, weights):
    wflat = lax.optimization_barrier(weights.reshape(V // 4, 128))
    table = wflat.reshape(V, D)
    idxf = lax.optimization_barrier(inputs.T.reshape(B * S // 128, 128))
    idx3 = idxf.reshape(S, B // 128, 128)

    mesh = plsc.VectorSubcoreMesh(core_axis_name="c", subcore_axis_name="s")

    p2 = pl.kernel(
        _p2_body,
        out_type=jax.ShapeDtypeStruct((S, D, B), jnp.float32),
        mesh=mesh,
        compiler_params=pltpu.CompilerParams(use_tc_tiling_on_sc=False,
                                             needs_layout_passes=False),
        scratch_types=[
            pltpu.VMEM((S, 4, 128), jnp.int32),
            pltpu.VMEM((BPW, D), jnp.float32),
            pltpu.VMEM((BPW, D), jnp.float32),
            pltpu.VMEM((D, BPW + 1), jnp.float32),
            pltpu.VMEM((D, BPW), jnp.float32),
            pltpu.VMEM((D, BPW), jnp.float32),
            pltpu.SemaphoreType.DMA,
            pltpu.SemaphoreType.DMA,
            pltpu.SemaphoreType.DMA,
            pltpu.SemaphoreType.DMA,
        ],
    )
    out3 = p2(idx3, table)

    return out3.transpose(2, 0, 1)
